# baseline (speedup 1.0000x reference)
import functools

import jax
import jax.numpy as jnp
from jax import lax
from jax.experimental import pallas as pl
from jax.experimental.pallas import tpu as pltpu
from jax.experimental.pallas import tpu_sc as plsc

F32 = jnp.float32
I32 = jnp.int32

_NC = 2
_NS = 16
_NW = _NC * _NS

_BLK = 256
_FFBLK = 1024


def _mm_body(x_ref, w_ref, b_ref, o_ref):
    o_ref[...] = (
        jnp.dot(x_ref[...], w_ref[...], preferred_element_type=F32) + b_ref[...]
    )


def _mm(x, w, b, bm=256):
    m, k = x.shape
    n = w.shape[1]
    return pl.pallas_call(
        _mm_body,
        grid=(m // bm,),
        in_specs=[
            pl.BlockSpec((bm, k), lambda i: (i, 0)),
            pl.BlockSpec((k, n), lambda i: (0, 0)),
            pl.BlockSpec((1, n), lambda i: (0, 0)),
        ],
        out_specs=pl.BlockSpec((bm, n), lambda i: (i, 0)),
        out_shape=jax.ShapeDtypeStruct((m, n), F32),
    )(x, w, b.reshape(1, n))


def _attn_body(q_ref, k_ref, v_ref, o_ref, *, scale):
    q = q_ref[0, 0] * scale
    k = k_ref[0, 0]
    v = v_ref[0, 0]
    logits = lax.dot_general(
        q, k, (((1,), (1,)), ((), ())), preferred_element_type=F32
    )
    m = jnp.max(logits, axis=-1, keepdims=True)
    p = jnp.exp(logits - m)
    s = jnp.sum(p, axis=-1, keepdims=True)
    o = jnp.dot(p, v, preferred_element_type=F32) / s
    o_ref[0, 0] = o


def _attention(q, k, v, bq=512):
    bq_dim, nh, qlen, hd = q.shape
    bb, _, s, _ = k.shape
    scale = 1.0 / (hd ** 0.5)
    q_ix = (lambda b, h, i: (b, h, i, 0)) if bq_dim > 1 else (
        lambda b, h, i: (0, h, i, 0))
    return pl.pallas_call(
        functools.partial(_attn_body, scale=scale),
        grid=(bb, nh, qlen // bq),
        in_specs=[
            pl.BlockSpec((1, 1, bq, hd), q_ix),
            pl.BlockSpec((1, 1, s, hd), lambda b, h, i: (b, h, 0, 0)),
            pl.BlockSpec((1, 1, s, hd), lambda b, h, i: (b, h, 0, 0)),
        ],
        out_specs=pl.BlockSpec((1, 1, bq, hd), lambda b, h, i: (b, h, i, 0)),
        out_shape=jax.ShapeDtypeStruct((bb, nh, qlen, hd), F32),
    )(q, k, v)


def _ln_core(att, wo, bo, res, g, bta):
    y = jnp.dot(att, wo, preferred_element_type=F32) + bo + res
    mu = jnp.mean(y, axis=-1, keepdims=True)
    var = jnp.mean(y * y, axis=-1, keepdims=True) - mu * mu
    return (y - mu) * lax.rsqrt(var + 1e-5) * g + bta


def _lnproj_body(att_ref, wo_ref, bo_ref, res_ref, g_ref, b_ref, o_ref):
    o_ref[...] = _ln_core(
        att_ref[...], wo_ref[...], bo_ref[...], res_ref[...], g_ref[...],
        b_ref[...])


def _lnproj_router_body(att_ref, wo_ref, bo_ref, res_ref, g_ref, b_ref,
                        gw_ref, gb_ref, o_ref, r_ref, *, ne):
    xn = _ln_core(att_ref[...], wo_ref[...], bo_ref[...], res_ref[...],
                  g_ref[...], b_ref[...])
    o_ref[...] = xn
    logits = jnp.dot(xn, gw_ref[...], preferred_element_type=F32) + gb_ref[...]
    bm, lanes = logits.shape
    lane = lax.broadcasted_iota(I32, (bm, lanes), 1)
    neg = jnp.float32(-1e30)
    lg = jnp.where(lane < ne, logits, neg)
    m1 = jnp.max(lg, axis=-1, keepdims=True)
    i1 = jnp.min(jnp.where(lg == m1, lane, lanes), axis=-1, keepdims=True)
    lg2 = jnp.where(lane == i1, neg, lg)
    m2 = jnp.max(lg2, axis=-1, keepdims=True)
    i2 = jnp.min(jnp.where(lg2 == m2, lane, lanes), axis=-1, keepdims=True)
    s1 = 1.0 / (1.0 + jnp.exp(m2 - m1))
    s2 = 1.0 - s1
    out = jnp.where(lane == 0, i1.astype(F32),
          jnp.where(lane == 1, i2.astype(F32),
          jnp.where(lane == 2, s1, s2)))
    r_ref[...] = out


def _lnproj(att, wo, bo, res, g, bta, gw=None, gb=None, ne=8, bm=256):
    t, d = att.shape
    args = [att, wo, bo.reshape(1, d), res, g.reshape(1, d),
            bta.reshape(1, d)]
    in_specs = [
        pl.BlockSpec((bm, d), lambda i: (i, 0)),
        pl.BlockSpec((d, d), lambda i: (0, 0)),
        pl.BlockSpec((1, d), lambda i: (0, 0)),
        pl.BlockSpec((bm, d), lambda i: (i, 0)),
        pl.BlockSpec((1, d), lambda i: (0, 0)),
        pl.BlockSpec((1, d), lambda i: (0, 0)),
    ]
    if gw is None:
        return pl.pallas_call(
            _lnproj_body,
            grid=(t // bm,),
            in_specs=in_specs,
            out_specs=pl.BlockSpec((bm, d), lambda i: (i, 0)),
            out_shape=jax.ShapeDtypeStruct((t, d), F32),
        )(*args)
    args += [gw, gb]
    in_specs += [
        pl.BlockSpec((d, 128), lambda i: (0, 0)),
        pl.BlockSpec((1, 128), lambda i: (0, 0)),
    ]
    return pl.pallas_call(
        functools.partial(_lnproj_router_body, ne=ne),
        grid=(t // bm,),
        in_specs=in_specs,
        out_specs=[
            pl.BlockSpec((bm, d), lambda i: (i, 0)),
            pl.BlockSpec((bm, 128), lambda i: (i, 0)),
        ],
        out_shape=[
            jax.ShapeDtypeStruct((t, d), F32),
            jax.ShapeDtypeStruct((t, 128), F32),
        ],
    )(*args)


def _sc_gather(table, idx, chunk):
    n, d = table.shape
    m = idx.shape[0]
    mpw = m // _NW
    offs = []
    o = 0
    while o < mpw:
        c = min(chunk, mpw - o)
        offs.append((o, c))
        o += c
    mesh = plsc.VectorSubcoreMesh(core_axis_name="c", subcore_axis_name="s")

    @functools.partial(
        pl.kernel,
        out_type=jax.ShapeDtypeStruct((m, d), F32),
        mesh=mesh,
        scratch_types=[
            pltpu.VMEM((mpw,), I32),
            pltpu.VMEM((chunk, d), F32),
            pltpu.SemaphoreType.DMA,
        ],
    )
    def gk(table_hbm, idx_hbm, out_hbm, idx_v, rows_v, sem):
        wid = lax.axis_index("s") * _NC + lax.axis_index("c")
        base = wid * mpw
        pltpu.sync_copy(idx_hbm.at[pl.ds(base, mpw)], idx_v)
        for off, c in offs:
            pltpu.async_copy(
                table_hbm.at[idx_v.at[pl.ds(off, c)]],
                rows_v.at[pl.ds(0, c)], sem).wait()
            pltpu.sync_copy(rows_v.at[pl.ds(0, c)],
                            out_hbm.at[pl.ds(base + off, c)])

    return gk(table, idx)


def _ffn_body(be_ref, nused_ref, xs_ref, w1_ref, b1_ref, w2_ref, b2_ref,
              y_ref, acc_ref):
    bb = pl.program_id(0)
    f = pl.program_id(1)
    nf = pl.num_programs(1)

    @pl.when(bb < nused_ref[0])
    def _():
        @pl.when(f == 0)
        def _():
            acc_ref[...] = jnp.zeros_like(acc_ref)

        h = jnp.maximum(
            jnp.dot(xs_ref[...], w1_ref[0], preferred_element_type=F32)
            + b1_ref[0], 0.0)
        acc_ref[...] += jnp.dot(h, w2_ref[0], preferred_element_type=F32)

        @pl.when(f == nf - 1)
        def _():
            y_ref[...] = acc_ref[...] + b2_ref[0]


def _ffn_grouped(xs, w1, b1, w2, b2, be, nused, nblk):
    npad, d = xs.shape
    e, _, ff = w1.shape
    nf = ff // _FFBLK
    grid_spec = pltpu.PrefetchScalarGridSpec(
        num_scalar_prefetch=2,
        grid=(nblk, nf),
        in_specs=[
            pl.BlockSpec((_BLK, d), lambda bb, f, be, nu: (bb, 0)),
            pl.BlockSpec((1, d, _FFBLK), lambda bb, f, be, nu: (be[bb], 0, f)),
            pl.BlockSpec((1, 1, _FFBLK), lambda bb, f, be, nu: (be[bb], 0, f)),
            pl.BlockSpec((1, _FFBLK, d), lambda bb, f, be, nu: (be[bb], f, 0)),
            pl.BlockSpec((1, 1, d), lambda bb, f, be, nu: (be[bb], 0, 0)),
        ],
        out_specs=pl.BlockSpec((_BLK, d), lambda bb, f, be, nu: (bb, 0)),
        scratch_shapes=[pltpu.VMEM((_BLK, d), F32)],
    )
    return pl.pallas_call(
        _ffn_body,
        grid_spec=grid_spec,
        out_shape=jax.ShapeDtypeStruct((npad, d), F32),
    )(be, nused, xs, w1, b1.reshape(e, 1, ff), w2, b2.reshape(e, 1, d))


def _combine_body(tgt_ref, r1_ref, r2_ref, rt_ref, o_ref):
    rt = rt_ref[...]
    s1 = rt[:, 2:3]
    s2 = rt[:, 3:4]
    o_ref[...] = tgt_ref[...] + s1 * r1_ref[...] + s2 * r2_ref[...]


def _combine(tgt, r1, r2, router, bm=256):
    t, d = tgt.shape
    return pl.pallas_call(
        _combine_body,
        grid=(t // bm,),
        in_specs=[
            pl.BlockSpec((bm, d), lambda i: (i, 0)),
            pl.BlockSpec((bm, d), lambda i: (i, 0)),
            pl.BlockSpec((bm, d), lambda i: (i, 0)),
            pl.BlockSpec((bm, 128), lambda i: (i, 0)),
        ],
        out_specs=pl.BlockSpec((bm, d), lambda i: (i, 0)),
        out_shape=jax.ShapeDtypeStruct((t, d), F32),
    )(tgt, r1, r2, router)


def kernel(out, memory, embed, sa_wq, sa_bq, sa_wk, sa_bk, sa_wv, sa_bv,
           sa_wo, sa_bo, ca_wq, ca_bq, ca_wk, ca_bk, ca_wv, ca_bv, ca_wo,
           ca_bo, norm1_g, norm1_b, norm2_g, norm2_b, gate_w, gate_b, w1, b1,
           w2, b2):
    s, b, d = memory.shape
    q = embed.shape[0]
    e, _, ff = w1.shape
    nh = 16
    hd = d // nh
    t = b * q

    def heads(x, bb, ll):
        return x.reshape(bb, ll, nh, hd).transpose(0, 2, 1, 3)

    memf = memory.transpose(1, 0, 2).reshape(b * s, d)
    q_sa = _mm(embed, sa_wq, sa_bq)
    kv_sa = _mm(memf, jnp.concatenate([sa_wk, sa_wv], axis=1),
                jnp.concatenate([sa_bk, sa_bv]))
    k_sa = heads(kv_sa[:, :d], b, s)
    v_sa = heads(kv_sa[:, d:], b, s)
    q_sa_h = q_sa.reshape(1, q, nh, hd).transpose(0, 2, 1, 3)
    att1 = _attention(q_sa_h, k_sa, v_sa)
    att1f = att1.transpose(0, 2, 1, 3).reshape(t, d)
    embed2 = jnp.tile(embed, (b, 1))
    tgt1 = _lnproj(att1f, sa_wo, sa_bo, embed2, norm1_g, norm1_b)

    outf = out.transpose(1, 0, 2).reshape(b * s, d)
    q_ca = heads(_mm(tgt1, ca_wq, ca_bq), b, q)
    kv_ca = _mm(outf, jnp.concatenate([ca_wk, ca_wv], axis=1),
                jnp.concatenate([ca_bk, ca_bv]))
    k_ca = heads(kv_ca[:, :d], b, s)
    v_ca = heads(kv_ca[:, d:], b, s)
    att2 = _attention(q_ca, k_ca, v_ca)
    att2f = att2.transpose(0, 2, 1, 3).reshape(t, d)
    gw_pad = jnp.zeros((d, 128), F32).at[:, :e].set(gate_w)
    gb_pad = jnp.zeros((1, 128), F32).at[0, :e].set(gate_b)
    tgt2, router = _lnproj(att2f, ca_wo, ca_bo, tgt1, norm2_g, norm2_b,
                           gw=gw_pad, gb=gb_pad, ne=e)

    i1 = router[:, 0].astype(I32)
    i2 = router[:, 1].astype(I32)
    ek = jnp.concatenate([i1, i2])
    na = 2 * t
    perm = jnp.argsort(ek, stable=True)
    counts = jnp.bincount(ek, length=e)
    nblocks_e = (counts + _BLK - 1) // _BLK
    nb_cum = jnp.cumsum(nblocks_e)
    nused = nb_cum[-1]
    bstart_e = nb_cum - nblocks_e
    c_cum_ex = jnp.cumsum(counts) - counts
    padbefore = bstart_e * _BLK - c_cum_ex
    inv = jnp.zeros((na,), I32).at[perm].set(jnp.arange(na, dtype=I32))
    pos_a = inv + padbefore[ek]
    nblk = (na // _BLK) + e - 1
    npad = nblk * _BLK
    tokens = jnp.concatenate(
        [jnp.arange(t, dtype=I32), jnp.arange(t, dtype=I32)])
    slots = jnp.zeros((npad,), I32).at[pos_a].set(tokens)
    block_expert = jnp.minimum(
        jnp.searchsorted(nb_cum, jnp.arange(nblk, dtype=I32), side="right"),
        e - 1).astype(I32)

    xs = _sc_gather(tgt2, slots, chunk=96)
    y = _ffn_grouped(xs, w1, b1, w2, b2, block_expert,
                     nused.reshape(1), nblk)
    r = _sc_gather(y, pos_a, chunk=64)
    outk = _combine(tgt2, r[:t], r[t:], router)
    return outk.reshape(b, q, d).transpose(1, 0, 2)


# speedup vs baseline: 1.0156x; 1.0156x over previous
import functools

import jax
import jax.numpy as jnp
from jax import lax
from jax.experimental import pallas as pl
from jax.experimental.pallas import tpu as pltpu
from jax.experimental.pallas import tpu_sc as plsc

F32 = jnp.float32
BF16 = jnp.bfloat16
I32 = jnp.int32

_NC = 2
_NS = 16
_NW = _NC * _NS

_BLK = 256
_FFBLK = 1024


_PREC = None


def _mm_body(x_ref, w_ref, b_ref, o_ref):
    o_ref[...] = (
        jnp.dot(x_ref[...], w_ref[...], preferred_element_type=F32,
                precision=_PREC) + b_ref[...]
    )


def _mm(x, w, b, bm=256):
    m, k = x.shape
    n = w.shape[1]
    return pl.pallas_call(
        _mm_body,
        grid=(m // bm,),
        in_specs=[
            pl.BlockSpec((bm, k), lambda i: (i, 0)),
            pl.BlockSpec((k, n), lambda i: (0, 0)),
            pl.BlockSpec((1, n), lambda i: (0, 0)),
        ],
        out_specs=pl.BlockSpec((bm, n), lambda i: (i, 0)),
        out_shape=jax.ShapeDtypeStruct((m, n), F32),
    )(x, w, b.reshape(1, n))


def _attn_body(q_ref, k_ref, v_ref, o_ref, *, scale):
    q = q_ref[0, 0] * scale
    k = k_ref[0, 0]
    v = v_ref[0, 0]
    logits = lax.dot_general(
        q, k, (((1,), (1,)), ((), ())), preferred_element_type=F32,
        precision=_PREC
    )
    m = jnp.max(logits, axis=-1, keepdims=True)
    p = jnp.exp(logits - m)
    s = jnp.sum(p, axis=-1, keepdims=True)
    p = p / s
    o_ref[0, 0] = jnp.dot(p, v, preferred_element_type=F32, precision=_PREC)


def _attention(q, k, v, bq=512):
    bq_dim, nh, qlen, hd = q.shape
    bb, _, s, _ = k.shape
    scale = 1.0 / (hd ** 0.5)
    q_ix = (lambda b, h, i: (b, h, i, 0)) if bq_dim > 1 else (
        lambda b, h, i: (0, h, i, 0))
    return pl.pallas_call(
        functools.partial(_attn_body, scale=scale),
        grid=(bb, nh, qlen // bq),
        in_specs=[
            pl.BlockSpec((1, 1, bq, hd), q_ix),
            pl.BlockSpec((1, 1, s, hd), lambda b, h, i: (b, h, 0, 0)),
            pl.BlockSpec((1, 1, s, hd), lambda b, h, i: (b, h, 0, 0)),
        ],
        out_specs=pl.BlockSpec((1, 1, bq, hd), lambda b, h, i: (b, h, i, 0)),
        out_shape=jax.ShapeDtypeStruct((bb, nh, qlen, hd), F32),
    )(q, k, v)


def _ln_core(att, wo, bo, res, g, bta):
    y = jnp.dot(att, wo, preferred_element_type=F32, precision=_PREC) \
        + bo + res
    mu = jnp.mean(y, axis=-1, keepdims=True)
    yc = y - mu
    var = jnp.mean(yc * yc, axis=-1, keepdims=True)
    return yc / jnp.sqrt(var + 1e-5) * g + bta


def _lnproj_body(att_ref, wo_ref, bo_ref, res_ref, g_ref, b_ref, o_ref):
    o_ref[...] = _ln_core(
        att_ref[...], wo_ref[...], bo_ref[...], res_ref[...], g_ref[...],
        b_ref[...])


def _lnproj(att, wo, bo, res, g, bta, bm=256):
    t, d = att.shape
    args = [att, wo, bo.reshape(1, d), res, g.reshape(1, d),
            bta.reshape(1, d)]
    in_specs = [
        pl.BlockSpec((bm, d), lambda i: (i, 0)),
        pl.BlockSpec((d, d), lambda i: (0, 0)),
        pl.BlockSpec((1, d), lambda i: (0, 0)),
        pl.BlockSpec((bm, d), lambda i: (i, 0)),
        pl.BlockSpec((1, d), lambda i: (0, 0)),
        pl.BlockSpec((1, d), lambda i: (0, 0)),
    ]
    return pl.pallas_call(
        _lnproj_body,
        grid=(t // bm,),
        in_specs=in_specs,
        out_specs=pl.BlockSpec((bm, d), lambda i: (i, 0)),
        out_shape=jax.ShapeDtypeStruct((t, d), F32),
    )(*args)


def _sc_gather(table, idx, chunk):
    n, d = table.shape
    m = idx.shape[0]
    mpw = m // _NW
    offs = []
    o = 0
    while o < mpw:
        c = min(chunk, mpw - o)
        offs.append((o, c))
        o += c
    mesh = plsc.VectorSubcoreMesh(core_axis_name="c", subcore_axis_name="s")

    @functools.partial(
        pl.kernel,
        out_type=jax.ShapeDtypeStruct((m, d), F32),
        mesh=mesh,
        scratch_types=[
            pltpu.VMEM((mpw,), I32),
            pltpu.VMEM((chunk, d), F32),
            pltpu.SemaphoreType.DMA,
        ],
    )
    def gk(table_hbm, idx_hbm, out_hbm, idx_v, rows_v, sem):
        wid = lax.axis_index("s") * _NC + lax.axis_index("c")
        base = wid * mpw
        pltpu.sync_copy(idx_hbm.at[pl.ds(base, mpw)], idx_v)
        for off, c in offs:
            pltpu.async_copy(
                table_hbm.at[idx_v.at[pl.ds(off, c)]],
                rows_v.at[pl.ds(0, c)], sem).wait()
            pltpu.sync_copy(rows_v.at[pl.ds(0, c)],
                            out_hbm.at[pl.ds(base + off, c)])

    return gk(table, idx)


def _ffn_body(be_ref, nused_ref, xs_ref, w1_ref, b1_ref, w2_ref, b2_ref,
              y_ref, acc_ref):
    bb = pl.program_id(0)
    f = pl.program_id(1)
    nf = pl.num_programs(1)

    @pl.when(bb < nused_ref[0])
    def _():
        @pl.when(f == 0)
        def _():
            acc_ref[...] = jnp.zeros_like(acc_ref)

        h = jnp.maximum(
            jnp.dot(xs_ref[...].astype(BF16), w1_ref[0],
                    preferred_element_type=F32) + b1_ref[0], 0.0)
        acc_ref[...] += jnp.dot(h.astype(BF16), w2_ref[0],
                                preferred_element_type=F32)

        @pl.when(f == nf - 1)
        def _():
            y_ref[...] = acc_ref[...] + b2_ref[0]


def _ffn_grouped(xs, w1, b1, w2, b2, be, nused, nblk):
    npad, d = xs.shape
    e, _, ff = w1.shape
    nf = ff // _FFBLK
    grid_spec = pltpu.PrefetchScalarGridSpec(
        num_scalar_prefetch=2,
        grid=(nblk, nf),
        in_specs=[
            pl.BlockSpec((_BLK, d), lambda bb, f, be, nu: (bb, 0)),
            pl.BlockSpec((1, d, _FFBLK), lambda bb, f, be, nu: (be[bb], 0, f)),
            pl.BlockSpec((1, 1, _FFBLK), lambda bb, f, be, nu: (be[bb], 0, f)),
            pl.BlockSpec((1, _FFBLK, d), lambda bb, f, be, nu: (be[bb], f, 0)),
            pl.BlockSpec((1, 1, d), lambda bb, f, be, nu: (be[bb], 0, 0)),
        ],
        out_specs=pl.BlockSpec((_BLK, d), lambda bb, f, be, nu: (bb, 0)),
        scratch_shapes=[pltpu.VMEM((_BLK, d), F32)],
    )
    return pl.pallas_call(
        _ffn_body,
        grid_spec=grid_spec,
        out_shape=jax.ShapeDtypeStruct((npad, d), F32),
    )(be, nused, xs, w1.astype(BF16), b1.reshape(e, 1, ff),
      w2.astype(BF16), b2.reshape(e, 1, d))


def _combine_body(tgt_ref, r1_ref, r2_ref, rt_ref, o_ref):
    rt = rt_ref[...]
    s1 = rt[:, 2:3]
    s2 = rt[:, 3:4]
    o_ref[...] = tgt_ref[...] + s1 * r1_ref[...] + s2 * r2_ref[...]


def _combine(tgt, r1, r2, router, bm=256):
    t, d = tgt.shape
    return pl.pallas_call(
        _combine_body,
        grid=(t // bm,),
        in_specs=[
            pl.BlockSpec((bm, d), lambda i: (i, 0)),
            pl.BlockSpec((bm, d), lambda i: (i, 0)),
            pl.BlockSpec((bm, d), lambda i: (i, 0)),
            pl.BlockSpec((bm, 128), lambda i: (i, 0)),
        ],
        out_specs=pl.BlockSpec((bm, d), lambda i: (i, 0)),
        out_shape=jax.ShapeDtypeStruct((t, d), F32),
    )(tgt, r1, r2, router)


def kernel(out, memory, embed, sa_wq, sa_bq, sa_wk, sa_bk, sa_wv, sa_bv,
           sa_wo, sa_bo, ca_wq, ca_bq, ca_wk, ca_bk, ca_wv, ca_bv, ca_wo,
           ca_bo, norm1_g, norm1_b, norm2_g, norm2_b, gate_w, gate_b, w1, b1,
           w2, b2):
    s, b, d = memory.shape
    q = embed.shape[0]
    e, _, ff = w1.shape
    nh = 16
    hd = d // nh
    t = b * q

    def heads(x, bb, ll):
        return x.reshape(bb, ll, nh, hd).transpose(0, 2, 1, 3)

    memf = memory.transpose(1, 0, 2).reshape(b * s, d)
    q_sa = _mm(embed, sa_wq, sa_bq)
    kv_sa = _mm(memf, jnp.concatenate([sa_wk, sa_wv], axis=1),
                jnp.concatenate([sa_bk, sa_bv]))
    k_sa = heads(kv_sa[:, :d], b, s)
    v_sa = heads(kv_sa[:, d:], b, s)
    q_sa_h = q_sa.reshape(1, q, nh, hd).transpose(0, 2, 1, 3)
    att1 = _attention(q_sa_h, k_sa, v_sa)
    att1f = att1.transpose(0, 2, 1, 3).reshape(t, d)
    embed2 = jnp.tile(embed, (b, 1))
    tgt1 = _lnproj(att1f, sa_wo, sa_bo, embed2, norm1_g, norm1_b)

    outf = out.transpose(1, 0, 2).reshape(b * s, d)
    q_ca = heads(_mm(tgt1, ca_wq, ca_bq), b, q)
    kv_ca = _mm(outf, jnp.concatenate([ca_wk, ca_wv], axis=1),
                jnp.concatenate([ca_bk, ca_bv]))
    k_ca = heads(kv_ca[:, :d], b, s)
    v_ca = heads(kv_ca[:, d:], b, s)
    att2 = _attention(q_ca, k_ca, v_ca)
    att2f = att2.transpose(0, 2, 1, 3).reshape(t, d)
    tgt2 = _lnproj(att2f, ca_wo, ca_bo, tgt1, norm2_g, norm2_b)

    logits = tgt2 @ gate_w + gate_b
    topv, topi = lax.top_k(logits, 2)
    scores = jax.nn.softmax(topv, axis=-1)
    router = jnp.pad(
        jnp.concatenate([topi.astype(F32), scores], axis=1),
        ((0, 0), (0, 124)))

    i1 = topi[:, 0].astype(I32)
    i2 = topi[:, 1].astype(I32)
    ek = jnp.concatenate([i1, i2])
    na = 2 * t
    ohm = (ek[:, None] == jnp.arange(e, dtype=I32)[None, :]).astype(I32)
    csum = jnp.cumsum(ohm, axis=0)
    counts = csum[-1]
    rank_a = jnp.sum(ohm * csum, axis=1) - 1
    nblocks_e = (counts + _BLK - 1) // _BLK
    nb_cum = jnp.cumsum(nblocks_e)
    nused = nb_cum[-1]
    bstart_e = nb_cum - nblocks_e
    pos_a = jnp.sum(ohm * bstart_e[None, :], axis=1) * _BLK + rank_a
    nblk = (na // _BLK) + e - 1
    npad = nblk * _BLK
    tokens = jnp.concatenate(
        [jnp.arange(t, dtype=I32), jnp.arange(t, dtype=I32)])
    slots = jnp.zeros((npad,), I32).at[pos_a].set(tokens)
    block_expert = jnp.minimum(
        jnp.sum((nb_cum[None, :] <=
                 jnp.arange(nblk, dtype=I32)[:, None]).astype(I32), axis=1),
        e - 1).astype(I32)

    xs = _sc_gather(tgt2, slots, chunk=96)
    y = _ffn_grouped(xs, w1, b1, w2, b2, block_expert,
                     nused.reshape(1), nblk)
    r = _sc_gather(y, pos_a, chunk=64)
    outk = _combine(tgt2, r[:t], r[t:], router)
    return outk.reshape(b, q, d).transpose(1, 0, 2)


# speedup vs baseline: 1.0872x; 1.0705x over previous
import functools

import jax
import jax.numpy as jnp
from jax import lax
from jax.experimental import pallas as pl
from jax.experimental.pallas import tpu as pltpu
from jax.experimental.pallas import tpu_sc as plsc

F32 = jnp.float32
BF16 = jnp.bfloat16
I32 = jnp.int32

_NC = 2
_NS = 16
_NW = _NC * _NS

_BLK = 256
_FFBLK = 1024


_PREC = None


def _mm_body(x_ref, w_ref, b_ref, o_ref):
    o_ref[...] = (
        jnp.dot(x_ref[...], w_ref[...], preferred_element_type=F32,
                precision=_PREC) + b_ref[...]
    )


def _mm(x, w, b, bm=256):
    m, k = x.shape
    n = w.shape[1]
    return pl.pallas_call(
        _mm_body,
        grid=(m // bm,),
        in_specs=[
            pl.BlockSpec((bm, k), lambda i: (i, 0)),
            pl.BlockSpec((k, n), lambda i: (0, 0)),
            pl.BlockSpec((1, n), lambda i: (0, 0)),
        ],
        out_specs=pl.BlockSpec((bm, n), lambda i: (i, 0)),
        out_shape=jax.ShapeDtypeStruct((m, n), F32),
    )(x, w, b.reshape(1, n))


def _attn_body(q_ref, k_ref, v_ref, o_ref, *, scale):
    q = q_ref[0, 0] * scale
    k = k_ref[0, 0]
    v = v_ref[0, 0]
    logits = lax.dot_general(
        q, k, (((1,), (1,)), ((), ())), preferred_element_type=F32,
        precision=_PREC
    )
    m = jnp.max(logits, axis=-1, keepdims=True)
    p = jnp.exp(logits - m)
    s = jnp.sum(p, axis=-1, keepdims=True)
    p = p / s
    o_ref[0, 0] = jnp.dot(p, v, preferred_element_type=F32, precision=_PREC)


def _attention(q, k, v, bq=512):
    bq_dim, nh, qlen, hd = q.shape
    bb, _, s, _ = k.shape
    scale = 1.0 / (hd ** 0.5)
    q_ix = (lambda b, h, i: (b, h, i, 0)) if bq_dim > 1 else (
        lambda b, h, i: (0, h, i, 0))
    return pl.pallas_call(
        functools.partial(_attn_body, scale=scale),
        grid=(bb, nh, qlen // bq),
        in_specs=[
            pl.BlockSpec((1, 1, bq, hd), q_ix),
            pl.BlockSpec((1, 1, s, hd), lambda b, h, i: (b, h, 0, 0)),
            pl.BlockSpec((1, 1, s, hd), lambda b, h, i: (b, h, 0, 0)),
        ],
        out_specs=pl.BlockSpec((1, 1, bq, hd), lambda b, h, i: (b, h, i, 0)),
        out_shape=jax.ShapeDtypeStruct((bb, nh, qlen, hd), F32),
    )(q, k, v)


def _ln_core(att, wo, bo, res, g, bta):
    y = jnp.dot(att, wo, preferred_element_type=F32, precision=_PREC) \
        + bo + res
    mu = jnp.mean(y, axis=-1, keepdims=True)
    yc = y - mu
    var = jnp.mean(yc * yc, axis=-1, keepdims=True)
    return yc / jnp.sqrt(var + 1e-5) * g + bta


def _lnproj_body(att_ref, wo_ref, bo_ref, res_ref, g_ref, b_ref, o_ref):
    o_ref[...] = _ln_core(
        att_ref[...], wo_ref[...], bo_ref[...], res_ref[...], g_ref[...],
        b_ref[...])


def _lnproj(att, wo, bo, res, g, bta, bm=256):
    t, d = att.shape
    args = [att, wo, bo.reshape(1, d), res, g.reshape(1, d),
            bta.reshape(1, d)]
    in_specs = [
        pl.BlockSpec((bm, d), lambda i: (i, 0)),
        pl.BlockSpec((d, d), lambda i: (0, 0)),
        pl.BlockSpec((1, d), lambda i: (0, 0)),
        pl.BlockSpec((bm, d), lambda i: (i, 0)),
        pl.BlockSpec((1, d), lambda i: (0, 0)),
        pl.BlockSpec((1, d), lambda i: (0, 0)),
    ]
    return pl.pallas_call(
        _lnproj_body,
        grid=(t // bm,),
        in_specs=in_specs,
        out_specs=pl.BlockSpec((bm, d), lambda i: (i, 0)),
        out_shape=jax.ShapeDtypeStruct((t, d), F32),
    )(*args)


def _sc_gather(table, idx, chunk):
    n, d = table.shape
    m = idx.shape[0]
    mpw = m // _NW
    offs = []
    o = 0
    while o < mpw:
        c = min(chunk, mpw - o)
        offs.append((o, c))
        o += c
    mesh = plsc.VectorSubcoreMesh(core_axis_name="c", subcore_axis_name="s")

    @functools.partial(
        pl.kernel,
        out_type=jax.ShapeDtypeStruct((m, d), F32),
        mesh=mesh,
        scratch_types=[
            pltpu.VMEM((mpw,), I32),
            pltpu.VMEM((chunk, d), F32),
            pltpu.SemaphoreType.DMA,
        ],
    )
    def gk(table_hbm, idx_hbm, out_hbm, idx_v, rows_v, sem):
        wid = lax.axis_index("s") * _NC + lax.axis_index("c")
        base = wid * mpw
        pltpu.sync_copy(idx_hbm.at[pl.ds(base, mpw)], idx_v)
        for off, c in offs:
            pltpu.async_copy(
                table_hbm.at[idx_v.at[pl.ds(off, c)]],
                rows_v.at[pl.ds(0, c)], sem).wait()
            pltpu.sync_copy(rows_v.at[pl.ds(0, c)],
                            out_hbm.at[pl.ds(base + off, c)])

    return gk(table, idx)


def _ffn_body(be_ref, nused_ref, xs_ref, w1_ref, b1_ref, w2_ref, b2_ref,
              y_ref, acc_ref):
    bb = pl.program_id(0)
    f = pl.program_id(1)
    nf = pl.num_programs(1)

    @pl.when(bb < nused_ref[0])
    def _():
        @pl.when(f == 0)
        def _():
            acc_ref[...] = jnp.zeros_like(acc_ref)

        h = jnp.maximum(
            jnp.dot(xs_ref[...].astype(BF16), w1_ref[0],
                    preferred_element_type=F32) + b1_ref[0], 0.0)
        acc_ref[...] += jnp.dot(h.astype(BF16), w2_ref[0],
                                preferred_element_type=F32)

        @pl.when(f == nf - 1)
        def _():
            y_ref[...] = acc_ref[...] + b2_ref[0]


def _ffn_grouped(xs, w1, b1, w2, b2, be, nused, nblk):
    npad, d = xs.shape
    e, _, ff = w1.shape
    nf = ff // _FFBLK
    grid_spec = pltpu.PrefetchScalarGridSpec(
        num_scalar_prefetch=2,
        grid=(nblk, nf),
        in_specs=[
            pl.BlockSpec((_BLK, d), lambda bb, f, be, nu: (bb, 0)),
            pl.BlockSpec((1, d, _FFBLK), lambda bb, f, be, nu: (be[bb], 0, f)),
            pl.BlockSpec((1, 1, _FFBLK), lambda bb, f, be, nu: (be[bb], 0, f)),
            pl.BlockSpec((1, _FFBLK, d), lambda bb, f, be, nu: (be[bb], f, 0)),
            pl.BlockSpec((1, 1, d), lambda bb, f, be, nu: (be[bb], 0, 0)),
        ],
        out_specs=pl.BlockSpec((_BLK, d), lambda bb, f, be, nu: (bb, 0)),
        scratch_shapes=[pltpu.VMEM((_BLK, d), F32)],
    )
    return pl.pallas_call(
        _ffn_body,
        grid_spec=grid_spec,
        out_shape=jax.ShapeDtypeStruct((npad, d), F32),
    )(be, nused, xs, w1.astype(BF16), b1.reshape(e, 1, ff),
      w2.astype(BF16), b2.reshape(e, 1, d))


def _combine_body(tgt_ref, r1_ref, r2_ref, rt_ref, o_ref):
    rt = rt_ref[...]
    s1 = rt[:, 2:3]
    s2 = rt[:, 3:4]
    o_ref[...] = tgt_ref[...] + s1 * r1_ref[...] + s2 * r2_ref[...]


def _combine(tgt, r1, r2, router, bm=256):
    t, d = tgt.shape
    return pl.pallas_call(
        _combine_body,
        grid=(t // bm,),
        in_specs=[
            pl.BlockSpec((bm, d), lambda i: (i, 0)),
            pl.BlockSpec((bm, d), lambda i: (i, 0)),
            pl.BlockSpec((bm, d), lambda i: (i, 0)),
            pl.BlockSpec((bm, 128), lambda i: (i, 0)),
        ],
        out_specs=pl.BlockSpec((bm, d), lambda i: (i, 0)),
        out_shape=jax.ShapeDtypeStruct((t, d), F32),
    )(tgt, r1, r2, router)


def kernel(out, memory, embed, sa_wq, sa_bq, sa_wk, sa_bk, sa_wv, sa_bv,
           sa_wo, sa_bo, ca_wq, ca_bq, ca_wk, ca_bk, ca_wv, ca_bv, ca_wo,
           ca_bo, norm1_g, norm1_b, norm2_g, norm2_b, gate_w, gate_b, w1, b1,
           w2, b2):
    s, b, d = memory.shape
    q = embed.shape[0]
    e, _, ff = w1.shape
    nh = 16
    hd = d // nh
    t = b * q

    def heads(x, bb, ll):
        return x.reshape(bb, ll, nh, hd).transpose(0, 2, 1, 3)

    memf = memory.transpose(1, 0, 2).reshape(b * s, d)
    q_sa = _mm(embed, sa_wq, sa_bq)
    kv_sa = _mm(memf, jnp.concatenate([sa_wk, sa_wv], axis=1),
                jnp.concatenate([sa_bk, sa_bv]))
    k_sa = heads(kv_sa[:, :d], b, s)
    v_sa = heads(kv_sa[:, d:], b, s)
    q_sa_h = q_sa.reshape(1, q, nh, hd).transpose(0, 2, 1, 3)
    att1 = _attention(q_sa_h, k_sa, v_sa)
    att1f = att1.transpose(0, 2, 1, 3).reshape(t, d)
    embed2 = jnp.tile(embed, (b, 1))
    tgt1 = _lnproj(att1f, sa_wo, sa_bo, embed2, norm1_g, norm1_b)

    outf = out.transpose(1, 0, 2).reshape(b * s, d)
    q_ca = heads(_mm(tgt1, ca_wq, ca_bq), b, q)
    kv_ca = _mm(outf, jnp.concatenate([ca_wk, ca_wv], axis=1),
                jnp.concatenate([ca_bk, ca_bv]))
    k_ca = heads(kv_ca[:, :d], b, s)
    v_ca = heads(kv_ca[:, d:], b, s)
    att2 = _attention(q_ca, k_ca, v_ca)
    att2f = att2.transpose(0, 2, 1, 3).reshape(t, d)
    tgt2 = _lnproj(att2f, ca_wo, ca_bo, tgt1, norm2_g, norm2_b)

    logits = tgt2 @ gate_w + gate_b
    topv, topi = lax.top_k(logits, 2)
    scores = jax.nn.softmax(topv, axis=-1)
    router = jnp.pad(
        jnp.concatenate([topi.astype(F32), scores], axis=1),
        ((0, 0), (0, 124)))

    i1 = topi[:, 0].astype(I32)
    i2 = topi[:, 1].astype(I32)
    ek = jnp.concatenate([i1, i2])
    na = 2 * t
    ohm = (ek[:, None] == jnp.arange(e, dtype=I32)[None, :]).astype(I32)
    csum = jnp.cumsum(ohm, axis=0)
    counts = csum[-1]
    rank_a = jnp.sum(ohm * csum, axis=1) - 1
    nblocks_e = (counts + _BLK - 1) // _BLK
    nb_cum = jnp.cumsum(nblocks_e)
    nused = nb_cum[-1]
    bstart_e = nb_cum - nblocks_e
    pos_a = jnp.sum(ohm * bstart_e[None, :], axis=1) * _BLK + rank_a
    nblk = (na // _BLK) + e - 1
    npad = nblk * _BLK
    tokens = jnp.concatenate(
        [jnp.arange(t, dtype=I32), jnp.arange(t, dtype=I32)])
    slots = (jnp.arange(npad, dtype=I32) % t).at[pos_a].set(tokens)
    block_expert = jnp.minimum(
        jnp.sum((nb_cum[None, :] <=
                 jnp.arange(nblk, dtype=I32)[:, None]).astype(I32), axis=1),
        e - 1).astype(I32)

    xs = _sc_gather(tgt2, slots, chunk=96)
    y = _ffn_grouped(xs, w1, b1, w2, b2, block_expert,
                     nused.reshape(1), nblk)
    r = _sc_gather(y, pos_a, chunk=64)
    outk = _combine(tgt2, r[:t], r[t:], router)
    return outk.reshape(b, q, d).transpose(1, 0, 2)
